# baseline (speedup 1.0000x reference)
import functools

import jax
import jax.numpy as jnp
from jax import lax
from jax.experimental import pallas as pl
from jax.experimental.pallas import tpu as pltpu
from jax.experimental.pallas import tpu_sc as plsc

N_CAS = 2000
N_USER = 8000
N = N_CAS + N_USER
E = 320000
D = 128
B = 4096
LAYERS = 3

NC = 2
NS = 16
NW = NC * NS
C = 80
N_PAD = 10240
ROWS_PER_SUB = N_PAD // NS
ROWS_PER_W = N_PAD // NW
K0 = 208
K1 = 48
KMAX = K0
OUT_ROWS_PER_W = 2 * B // NW
CF = 128
DSEG = D // 16

_mesh = plsc.VectorSubcoreMesh(core_axis_name="c", subcore_axis_name="s")


def _wid():
    return lax.axis_index("c") * NS + lax.axis_index("s")


@functools.partial(
    pl.kernel,
    out_type=(
        jax.ShapeDtypeStruct((N_PAD, D), jnp.float32),
        jax.ShapeDtypeStruct((N_PAD, D), jnp.float32),
    ),
    mesh=_mesh,
    scratch_types=(
        [pltpu.VMEM_SHARED((N_PAD, D), jnp.float32)]
        + [pltpu.VMEM((3, C), jnp.int32) for _ in range(8)]
        + [pltpu.VMEM((C, D), jnp.float32) for _ in range(4)]
        + [pltpu.SemaphoreType.DMA] * 16
    ),
    compiler_params=pltpu.CompilerParams(use_tc_tiling_on_sc=False,
                                        needs_layout_passes=False),
)
def _edge_layer(e_hbm, ed_hbm, p0_out, p1_out, acc,
                x0, x1, x2, x3, x4, x5, x6, x7,
                rb0, rb1, rb2, rb3,
                sg0, sg1, sg2, sg3, ss0, ss1, ss2, ss3,
                si0, si1, si2, si3, si4, si5, si6, si7):
    c = lax.axis_index("c")
    s = lax.axis_index("s")
    wid = c * NS + s
    r0 = s * ROWS_PER_SUB
    idxs = (x0, x1, x2, x3, x4, x5, x6, x7)
    rows = (rb0, rb1, rb2, rb3)
    gsem = (sg0, sg1, sg2, sg3)
    ssem = (ss0, ss1, ss2, ss3)
    isem = (si0, si1, si2, si3, si4, si5, si6, si7)
    nch = jnp.where(c == 0, K0, K1)
    ngrp = jnp.where(c == 0, K0 // 8, K1 // 8)

    def zfill(i, carry):
        for j in range(DSEG):
            rb0[i, pl.ds(j * 16, 16)] = jnp.zeros((16,), jnp.float32)
        return carry

    lax.fori_loop(0, 64, zfill, 0)

    def zcp(t, carry):
        pltpu.sync_copy(rb0.at[pl.ds(0, 64)], acc.at[pl.ds(r0 + t * 64, 64)])
        return carry

    lax.fori_loop(0, ROWS_PER_SUB // 64, zcp, 0)

    for j in range(6):
        pltpu.async_copy(ed_hbm.at[wid, j], idxs[j], isem[j])
    for j in range(3):
        pltpu.make_async_copy(ed_hbm.at[wid, 0], idxs[j], isem[j]).wait()
        pltpu.async_copy(e_hbm.at[idxs[j].at[0]], rows[j], gsem[j])
    plsc.subcore_barrier()

    NG = C // 16

    def do_chunk(k, i):
        b = i % 4
        b3 = (i + 3) % 4
        q = i % 8
        q3 = (i + 3) % 8
        q6 = (i + 6) % 8
        rcur = rows[b]
        xq = idxs[q]
        pltpu.make_async_copy(e_hbm.at[pl.ds(0, C)], rcur, gsem[b]).wait()

        def grp(gi, carry):
            w16 = plsc.bitcast(xq[2, pl.ds(gi * 16, 16)], jnp.float32)
            base_row = gi * 16
            for l in range(16):
                w = w16[l]
                for j in range(DSEG):
                    sl = pl.ds(j * 16, 16)
                    rcur[base_row + l, sl] = rcur[base_row + l, sl] * w
            return carry

        lax.fori_loop(0, NG // 2, grp, 0)

        @pl.when(k >= 1)
        def _():
            pltpu.make_async_copy(e_hbm.at[pl.ds(0, C)], rows[b3],
                                  ssem[b3]).wait()

        @pl.when(k + 3 < nch)
        def _():
            pltpu.make_async_copy(ed_hbm.at[wid, 0], idxs[q3],
                                  isem[q3]).wait()
            pltpu.async_copy(e_hbm.at[idxs[q3].at[0]], rows[b3], gsem[b3])

        @pl.when(k + 6 < nch)
        def _():
            pltpu.async_copy(ed_hbm.at[wid, k + 6], idxs[q6], isem[q6])

        lax.fori_loop(NG // 2, NG, grp, 0)
        pltpu.async_copy(rcur, acc.at[xq.at[1]], ssem[b], add=True)

    def grp8(g, carry):
        k = g * 8
        for i in range(8):
            do_chunk(k + i, i)
        return carry

    lax.fori_loop(0, ngrp, grp8, 0)
    pltpu.make_async_copy(e_hbm.at[pl.ds(0, C)], rb3, ss3).wait()
    plsc.subcore_barrier()

    @pl.when(c == 0)
    def _():
        def wb(t, carry):
            pltpu.sync_copy(acc.at[pl.ds(r0 + t * 64, 64)],
                            rb0.at[pl.ds(0, 64)])
            pltpu.sync_copy(rb0.at[pl.ds(0, 64)],
                            p0_out.at[pl.ds(r0 + t * 64, 64)])
            return carry

        lax.fori_loop(0, ROWS_PER_SUB // 64, wb, 0)

    @pl.when(c == 1)
    def _():
        def wb(t, carry):
            pltpu.sync_copy(acc.at[pl.ds(r0 + t * 64, 64)],
                            rb0.at[pl.ds(0, 64)])
            pltpu.sync_copy(rb0.at[pl.ds(0, 64)],
                            p1_out.at[pl.ds(r0 + t * 64, 64)])
            return carry

        lax.fori_loop(0, ROWS_PER_SUB // 64, wb, 0)


@functools.partial(
    pl.kernel,
    out_type=(
        jax.ShapeDtypeStruct((N_PAD, D), jnp.float32),
        jax.ShapeDtypeStruct((N_PAD, D), jnp.float32),
    ),
    mesh=_mesh,
    scratch_types=[
        pltpu.VMEM((ROWS_PER_W, D), jnp.float32),
        pltpu.VMEM((ROWS_PER_W, D), jnp.float32),
        pltpu.VMEM((ROWS_PER_W, D), jnp.float32),
    ],
)
def _combine(p0_hbm, p1_hbm, s_hbm, e_out, s_out, p0v, p1v, sv):
    wid = _wid()
    r0 = wid * ROWS_PER_W
    pltpu.sync_copy(p0_hbm.at[pl.ds(r0, ROWS_PER_W)], p0v)
    pltpu.sync_copy(p1_hbm.at[pl.ds(r0, ROWS_PER_W)], p1v)
    pltpu.sync_copy(s_hbm.at[pl.ds(r0, ROWS_PER_W)], sv)

    def row(i, carry):
        for j in range(DSEG):
            sl = pl.ds(j * 16, 16)
            e = p0v[i, sl] + p1v[i, sl]
            p0v[i, sl] = e
            sv[i, sl] = sv[i, sl] + e
        return carry

    lax.fori_loop(0, ROWS_PER_W, row, 0)
    pltpu.sync_copy(p0v, e_out.at[pl.ds(r0, ROWS_PER_W)])
    pltpu.sync_copy(sv, s_out.at[pl.ds(r0, ROWS_PER_W)])


@functools.partial(
    pl.kernel,
    out_type=jax.ShapeDtypeStruct((2 * B, D), jnp.float32),
    mesh=_mesh,
    scratch_types=[
        pltpu.VMEM((CF,), jnp.int32),
        pltpu.VMEM((CF, D), jnp.float32),
        pltpu.VMEM((CF, D), jnp.float32),
        pltpu.VMEM((CF, D), jnp.float32),
        pltpu.SemaphoreType.DMA,
    ],
)
def _final_gather(p0_hbm, p1_hbm, s_hbm, g_hbm, out_hbm,
                  gidx, rsv, r0v, r1v, sem):
    wid = _wid()
    for t in range(OUT_ROWS_PER_W // CF):
        off = wid * OUT_ROWS_PER_W + t * CF
        pltpu.sync_copy(g_hbm.at[pl.ds(off, CF)], gidx)
        pltpu.async_copy(s_hbm.at[gidx], rsv, sem).wait()
        pltpu.async_copy(p0_hbm.at[gidx], r0v, sem).wait()
        pltpu.async_copy(p1_hbm.at[gidx], r1v, sem).wait()

        def row(i, carry):
            for j in range(DSEG):
                sl = pl.ds(j * 16, 16)
                rsv[i, sl] = (rsv[i, sl] + r0v[i, sl] + r1v[i, sl]) * 0.25
            return carry

        lax.fori_loop(0, CF, row, 0)
        pltpu.sync_copy(rsv, out_hbm.at[pl.ds(off, CF)])


def kernel(node_emb, edge_index, edge_weight, cas_idx, user_idx):
    src = edge_index[0]
    dst = edge_index[1]
    wbits = lax.bitcast_convert_type(edge_weight, jnp.int32)
    cap0 = NS * K0 * C
    cap1 = NS * K1 * C

    def part(x):
        xp = jnp.concatenate(
            [x, jnp.zeros((cap0 + cap1 - E,), jnp.int32)])
        a0 = xp[:cap0].reshape(NS, K0, C)
        a1 = xp[cap0:].reshape(NS, K1, C)
        a1 = jnp.pad(a1, ((0, 0), (0, K0 - K1), (0, 0)))
        return jnp.concatenate([a0, a1], axis=0)

    ed = jnp.stack([part(src), part(dst), part(wbits)], axis=2)
    e0 = jnp.zeros((N_PAD, D), jnp.float32).at[:N].set(node_emb)
    g = jnp.concatenate([cas_idx, user_idx + N_CAS]).astype(jnp.int32)

    p0, p1 = _edge_layer(e0, ed)
    e1, s1 = _combine(p0, p1, e0)
    p0, p1 = _edge_layer(e1, ed)
    e2, s2 = _combine(p0, p1, s1)
    p0, p1 = _edge_layer(e2, ed)
    return _final_gather(p0, p1, s2, g)


# speedup vs baseline: 1.7090x; 1.7090x over previous
import functools

import jax
import jax.numpy as jnp
from jax import lax
from jax.experimental import pallas as pl
from jax.experimental.pallas import tpu as pltpu
from jax.experimental.pallas import tpu_sc as plsc

N_CAS = 2000
N_USER = 8000
N = N_CAS + N_USER
E = 320000
D = 128
B = 4096
LAYERS = 3

NC = 2
NS = 16
NW = NC * NS
C = 96
N_PAD = 10240
ROWS_PER_SUB = N_PAD // NS
ROWS_PER_W = N_PAD // NW
K0 = 174
K1 = 36
KMAX = K0
OUT_ROWS_PER_W = 2 * B // NW
CF = 128
DSEG = D // 16

_mesh = plsc.VectorSubcoreMesh(core_axis_name="c", subcore_axis_name="s")


def _wid():
    return lax.axis_index("c") * NS + lax.axis_index("s")


@functools.partial(
    pl.kernel,
    out_type=(
        jax.ShapeDtypeStruct((N_PAD, D), jnp.float32),
        jax.ShapeDtypeStruct((N_PAD, D), jnp.float32),
    ),
    mesh=_mesh,
    scratch_types=(
        [pltpu.VMEM_SHARED((N_PAD, D), jnp.float32)]
        + [pltpu.VMEM((3, C), jnp.int32) for _ in range(6)]
        + [pltpu.VMEM((C, D), jnp.float32) for _ in range(3)]
        + [pltpu.SemaphoreType.DMA] * 12
    ),
    compiler_params=pltpu.CompilerParams(use_tc_tiling_on_sc=False,
                                        needs_layout_passes=False),
)
def _edge_layer(e_hbm, ed_hbm, p0_out, p1_out, acc,
                x0, x1, x2, x3, x4, x5,
                rb0, rb1, rb2,
                sg0, sg1, sg2, ss0, ss1, ss2,
                si0, si1, si2, si3, si4, si5):
    c = lax.axis_index("c")
    s = lax.axis_index("s")
    wid = c * NS + s
    r0 = s * ROWS_PER_SUB
    idxs = (x0, x1, x2, x3, x4, x5)
    rows = (rb0, rb1, rb2)
    gsem = (sg0, sg1, sg2)
    ssem = (ss0, ss1, ss2)
    isem = (si0, si1, si2, si3, si4, si5)
    nch = jnp.where(c == 0, K0, K1)
    ngrp = jnp.where(c == 0, K0 // 6, K1 // 6)

    def zfill(i, carry):
        for j in range(DSEG):
            rb0[i, pl.ds(j * 16, 16)] = jnp.zeros((16,), jnp.float32)
        return carry

    lax.fori_loop(0, 64, zfill, 0)

    def zcp(t, carry):
        pltpu.sync_copy(rb0.at[pl.ds(0, 64)], acc.at[pl.ds(r0 + t * 64, 64)])
        return carry

    lax.fori_loop(0, ROWS_PER_SUB // 64, zcp, 0)

    for j in range(4):
        pltpu.async_copy(ed_hbm.at[wid, j], idxs[j], isem[j])
    pltpu.make_async_copy(ed_hbm.at[wid, 0], idxs[0], isem[0]).wait()
    pltpu.async_copy(e_hbm.at[idxs[0].at[0]], rb0, sg0)
    pltpu.make_async_copy(ed_hbm.at[wid, 0], idxs[1], isem[1]).wait()
    pltpu.async_copy(e_hbm.at[idxs[1].at[0]], rb1, sg1)
    plsc.subcore_barrier()

    def do_chunk(k, i):
        b = i % 3
        b2 = (i + 2) % 3
        q = i % 6
        q2 = (i + 2) % 6
        q4 = (i + 4) % 6
        rcur = rows[b]
        xq = idxs[q]
        pltpu.make_async_copy(e_hbm.at[pl.ds(0, C)], rcur, gsem[b]).wait()

        def grp(gi, carry):
            w16 = plsc.bitcast(xq[2, pl.ds(gi * 16, 16)], jnp.float32)
            base_row = gi * 16
            for l in range(16):
                w = w16[l]
                for j in range(DSEG):
                    sl = pl.ds(j * 16, 16)
                    rcur[base_row + l, sl] = rcur[base_row + l, sl] * w
            return carry

        lax.fori_loop(0, 3, grp, 0)

        @pl.when(k >= 1)
        def _():
            pltpu.make_async_copy(e_hbm.at[pl.ds(0, C)], rows[b2],
                                  ssem[b2]).wait()

        @pl.when(k + 2 < nch)
        def _():
            pltpu.make_async_copy(ed_hbm.at[wid, 0], idxs[q2],
                                  isem[q2]).wait()
            pltpu.async_copy(e_hbm.at[idxs[q2].at[0]], rows[b2], gsem[b2])

        @pl.when(k + 4 < nch)
        def _():
            pltpu.async_copy(ed_hbm.at[wid, k + 4], idxs[q4], isem[q4])

        lax.fori_loop(3, 6, grp, 0)
        pltpu.async_copy(rcur, acc.at[xq.at[1]], ssem[b], add=True)

    def grp6(g, carry):
        k = g * 6
        for i in range(6):
            do_chunk(k + i, i)
        return carry

    lax.fori_loop(0, ngrp, grp6, 0)
    pltpu.make_async_copy(e_hbm.at[pl.ds(0, C)], rb2, ss2).wait()
    plsc.subcore_barrier()

    @pl.when(c == 0)
    def _():
        def wb(t, carry):
            pltpu.sync_copy(acc.at[pl.ds(r0 + t * 64, 64)],
                            rb0.at[pl.ds(0, 64)])
            pltpu.sync_copy(rb0.at[pl.ds(0, 64)],
                            p0_out.at[pl.ds(r0 + t * 64, 64)])
            return carry

        lax.fori_loop(0, ROWS_PER_SUB // 64, wb, 0)

    @pl.when(c == 1)
    def _():
        def wb(t, carry):
            pltpu.sync_copy(acc.at[pl.ds(r0 + t * 64, 64)],
                            rb0.at[pl.ds(0, 64)])
            pltpu.sync_copy(rb0.at[pl.ds(0, 64)],
                            p1_out.at[pl.ds(r0 + t * 64, 64)])
            return carry

        lax.fori_loop(0, ROWS_PER_SUB // 64, wb, 0)


@functools.partial(
    pl.kernel,
    out_type=(
        jax.ShapeDtypeStruct((N_PAD, D), jnp.float32),
        jax.ShapeDtypeStruct((N_PAD, D), jnp.float32),
    ),
    mesh=_mesh,
    scratch_types=[
        pltpu.VMEM((ROWS_PER_W, D), jnp.float32),
        pltpu.VMEM((ROWS_PER_W, D), jnp.float32),
        pltpu.VMEM((ROWS_PER_W, D), jnp.float32),
    ],
)
def _combine(p0_hbm, p1_hbm, s_hbm, e_out, s_out, p0v, p1v, sv):
    wid = _wid()
    r0 = wid * ROWS_PER_W
    pltpu.sync_copy(p0_hbm.at[pl.ds(r0, ROWS_PER_W)], p0v)
    pltpu.sync_copy(p1_hbm.at[pl.ds(r0, ROWS_PER_W)], p1v)
    pltpu.sync_copy(s_hbm.at[pl.ds(r0, ROWS_PER_W)], sv)

    def row(i, carry):
        for j in range(DSEG):
            sl = pl.ds(j * 16, 16)
            e = p0v[i, sl] + p1v[i, sl]
            p0v[i, sl] = e
            sv[i, sl] = sv[i, sl] + e
        return carry

    lax.fori_loop(0, ROWS_PER_W, row, 0)
    pltpu.sync_copy(p0v, e_out.at[pl.ds(r0, ROWS_PER_W)])
    pltpu.sync_copy(sv, s_out.at[pl.ds(r0, ROWS_PER_W)])


@functools.partial(
    pl.kernel,
    out_type=jax.ShapeDtypeStruct((2 * B, D), jnp.float32),
    mesh=_mesh,
    scratch_types=[
        pltpu.VMEM((CF,), jnp.int32),
        pltpu.VMEM((CF, D), jnp.float32),
        pltpu.VMEM((CF, D), jnp.float32),
        pltpu.VMEM((CF, D), jnp.float32),
        pltpu.SemaphoreType.DMA,
    ],
)
def _final_gather(p0_hbm, p1_hbm, s_hbm, g_hbm, out_hbm,
                  gidx, rsv, r0v, r1v, sem):
    wid = _wid()
    for t in range(OUT_ROWS_PER_W // CF):
        off = wid * OUT_ROWS_PER_W + t * CF
        pltpu.sync_copy(g_hbm.at[pl.ds(off, CF)], gidx)
        pltpu.async_copy(s_hbm.at[gidx], rsv, sem).wait()
        pltpu.async_copy(p0_hbm.at[gidx], r0v, sem).wait()
        pltpu.async_copy(p1_hbm.at[gidx], r1v, sem).wait()

        def row(i, carry):
            for j in range(DSEG):
                sl = pl.ds(j * 16, 16)
                rsv[i, sl] = (rsv[i, sl] + r0v[i, sl] + r1v[i, sl]) * 0.25
            return carry

        lax.fori_loop(0, CF, row, 0)
        pltpu.sync_copy(rsv, out_hbm.at[pl.ds(off, CF)])


def kernel(node_emb, edge_index, edge_weight, cas_idx, user_idx):
    src = edge_index[0]
    dst = edge_index[1]
    wbits = lax.bitcast_convert_type(edge_weight, jnp.int32)
    cap0 = NS * K0 * C
    cap1 = NS * K1 * C

    def part(x):
        xp = jnp.concatenate(
            [x, jnp.zeros((cap0 + cap1 - E,), jnp.int32)])
        a0 = xp[:cap0].reshape(NS, K0, C)
        a1 = xp[cap0:].reshape(NS, K1, C)
        a1 = jnp.pad(a1, ((0, 0), (0, K0 - K1), (0, 0)))
        return jnp.concatenate([a0, a1], axis=0)

    ed = jnp.stack([part(src), part(dst), part(wbits)], axis=2)
    e0 = jnp.zeros((N_PAD, D), jnp.float32).at[:N].set(node_emb)
    g = jnp.concatenate([cas_idx, user_idx + N_CAS]).astype(jnp.int32)

    p0, p1 = _edge_layer(e0, ed)
    e1, s1 = _combine(p0, p1, e0)
    p0, p1 = _edge_layer(e1, ed)
    e2, s2 = _combine(p0, p1, s1)
    p0, p1 = _edge_layer(e2, ed)
    return _final_gather(p0, p1, s2, g)


# speedup vs baseline: 1.8513x; 1.0832x over previous
import functools

import jax
import jax.numpy as jnp
from jax import lax
from jax.experimental import pallas as pl
from jax.experimental.pallas import tpu as pltpu
from jax.experimental.pallas import tpu_sc as plsc

N_CAS = 2000
N_USER = 8000
N = N_CAS + N_USER
E = 320000
D = 128
B = 4096
LAYERS = 3

NC = 2
NS = 16
NW = NC * NS
C = 96
N_PAD = 10240
ROWS_PER_SUB = N_PAD // NS
ROWS_PER_W = N_PAD // NW
K0 = 162
K1 = 48
KMAX = K0
OUT_ROWS_PER_W = 2 * B // NW
CF = 128
DSEG = D // 16

_mesh = plsc.VectorSubcoreMesh(core_axis_name="c", subcore_axis_name="s")


def _wid():
    return lax.axis_index("c") * NS + lax.axis_index("s")


@functools.partial(
    pl.kernel,
    out_type=(
        jax.ShapeDtypeStruct((N_PAD, D), jnp.float32),
        jax.ShapeDtypeStruct((N_PAD, D), jnp.float32),
    ),
    mesh=_mesh,
    scratch_types=(
        [pltpu.VMEM_SHARED((N_PAD, D), jnp.float32)]
        + [pltpu.VMEM((3, C), jnp.int32) for _ in range(6)]
        + [pltpu.VMEM((C, D), jnp.float32) for _ in range(3)]
        + [pltpu.SemaphoreType.DMA] * 12
    ),
    compiler_params=pltpu.CompilerParams(use_tc_tiling_on_sc=False,
                                        needs_layout_passes=False),
)
def _edge_layer(e_hbm, ed_hbm, p0_out, p1_out, acc,
                x0, x1, x2, x3, x4, x5,
                rb0, rb1, rb2,
                sg0, sg1, sg2, ss0, ss1, ss2,
                si0, si1, si2, si3, si4, si5):
    c = lax.axis_index("c")
    s = lax.axis_index("s")
    wid = c * NS + s
    r0 = s * ROWS_PER_SUB
    idxs = (x0, x1, x2, x3, x4, x5)
    rows = (rb0, rb1, rb2)
    gsem = (sg0, sg1, sg2)
    ssem = (ss0, ss1, ss2)
    isem = (si0, si1, si2, si3, si4, si5)
    nch = jnp.where(c == 0, K0, K1)
    ngrp = jnp.where(c == 0, K0 // 6, K1 // 6)

    def zfill(i, carry):
        for j in range(DSEG):
            rb0[i, pl.ds(j * 16, 16)] = jnp.zeros((16,), jnp.float32)
        return carry

    lax.fori_loop(0, 64, zfill, 0)

    def zcp(t, carry):
        pltpu.sync_copy(rb0.at[pl.ds(0, 64)], acc.at[pl.ds(r0 + t * 64, 64)])
        return carry

    lax.fori_loop(0, ROWS_PER_SUB // 64, zcp, 0)

    for j in range(4):
        pltpu.async_copy(ed_hbm.at[wid, j], idxs[j], isem[j])
    pltpu.make_async_copy(ed_hbm.at[wid, 0], idxs[0], isem[0]).wait()
    pltpu.async_copy(e_hbm.at[idxs[0].at[0]], rb0, sg0)
    pltpu.make_async_copy(ed_hbm.at[wid, 0], idxs[1], isem[1]).wait()
    pltpu.async_copy(e_hbm.at[idxs[1].at[0]], rb1, sg1)
    plsc.subcore_barrier()

    def do_chunk(k, i):
        b = i % 3
        b2 = (i + 2) % 3
        q = i % 6
        q2 = (i + 2) % 6
        q4 = (i + 4) % 6
        rcur = rows[b]
        xq = idxs[q]
        pltpu.make_async_copy(e_hbm.at[pl.ds(0, C)], rcur, gsem[b]).wait()

        def grp(gi, carry):
            w16 = plsc.bitcast(xq[2, pl.ds(gi * 16, 16)], jnp.float32)
            base_row = gi * 16
            for l in range(16):
                w = w16[l]
                for j in range(DSEG):
                    sl = pl.ds(j * 16, 16)
                    rcur[base_row + l, sl] = rcur[base_row + l, sl] * w
            return carry

        lax.fori_loop(0, 3, grp, 0)

        @pl.when(k >= 1)
        def _():
            pltpu.make_async_copy(e_hbm.at[pl.ds(0, C)], rows[b2],
                                  ssem[b2]).wait()

        @pl.when(k + 2 < nch)
        def _():
            pltpu.make_async_copy(ed_hbm.at[wid, 0], idxs[q2],
                                  isem[q2]).wait()
            pltpu.async_copy(e_hbm.at[idxs[q2].at[0]], rows[b2], gsem[b2])

        @pl.when(k + 4 < nch)
        def _():
            pltpu.async_copy(ed_hbm.at[wid, k + 4], idxs[q4], isem[q4])

        lax.fori_loop(3, 6, grp, 0)
        pltpu.async_copy(rcur, acc.at[xq.at[1]], ssem[b], add=True)

    def grp6(g, carry):
        k = g * 6
        for i in range(6):
            do_chunk(k + i, i)
        return carry

    lax.fori_loop(0, ngrp, grp6, 0)
    pltpu.make_async_copy(e_hbm.at[pl.ds(0, C)], rb2, ss2).wait()
    plsc.subcore_barrier()

    @pl.when(c == 0)
    def _():
        def wb(t, carry):
            pltpu.sync_copy(acc.at[pl.ds(r0 + t * 64, 64)],
                            rb0.at[pl.ds(0, 64)])
            pltpu.sync_copy(rb0.at[pl.ds(0, 64)],
                            p0_out.at[pl.ds(r0 + t * 64, 64)])
            return carry

        lax.fori_loop(0, ROWS_PER_SUB // 64, wb, 0)

    @pl.when(c == 1)
    def _():
        def wb(t, carry):
            pltpu.sync_copy(acc.at[pl.ds(r0 + t * 64, 64)],
                            rb0.at[pl.ds(0, 64)])
            pltpu.sync_copy(rb0.at[pl.ds(0, 64)],
                            p1_out.at[pl.ds(r0 + t * 64, 64)])
            return carry

        lax.fori_loop(0, ROWS_PER_SUB // 64, wb, 0)


@functools.partial(
    pl.kernel,
    out_type=(
        jax.ShapeDtypeStruct((N_PAD, D), jnp.float32),
        jax.ShapeDtypeStruct((N_PAD, D), jnp.float32),
    ),
    mesh=_mesh,
    scratch_types=[
        pltpu.VMEM((ROWS_PER_W, D), jnp.float32),
        pltpu.VMEM((ROWS_PER_W, D), jnp.float32),
        pltpu.VMEM((ROWS_PER_W, D), jnp.float32),
    ],
)
def _combine(p0_hbm, p1_hbm, s_hbm, e_out, s_out, p0v, p1v, sv):
    wid = _wid()
    r0 = wid * ROWS_PER_W
    pltpu.sync_copy(p0_hbm.at[pl.ds(r0, ROWS_PER_W)], p0v)
    pltpu.sync_copy(p1_hbm.at[pl.ds(r0, ROWS_PER_W)], p1v)
    pltpu.sync_copy(s_hbm.at[pl.ds(r0, ROWS_PER_W)], sv)

    def row(i, carry):
        for j in range(DSEG):
            sl = pl.ds(j * 16, 16)
            e = p0v[i, sl] + p1v[i, sl]
            p0v[i, sl] = e
            sv[i, sl] = sv[i, sl] + e
        return carry

    lax.fori_loop(0, ROWS_PER_W, row, 0)
    pltpu.sync_copy(p0v, e_out.at[pl.ds(r0, ROWS_PER_W)])
    pltpu.sync_copy(sv, s_out.at[pl.ds(r0, ROWS_PER_W)])


@functools.partial(
    pl.kernel,
    out_type=jax.ShapeDtypeStruct((2 * B, D), jnp.float32),
    mesh=_mesh,
    scratch_types=[
        pltpu.VMEM((CF,), jnp.int32),
        pltpu.VMEM((CF, D), jnp.float32),
        pltpu.VMEM((CF, D), jnp.float32),
        pltpu.VMEM((CF, D), jnp.float32),
        pltpu.SemaphoreType.DMA,
    ],
)
def _final_gather(p0_hbm, p1_hbm, s_hbm, g_hbm, out_hbm,
                  gidx, rsv, r0v, r1v, sem):
    wid = _wid()
    for t in range(OUT_ROWS_PER_W // CF):
        off = wid * OUT_ROWS_PER_W + t * CF
        pltpu.sync_copy(g_hbm.at[pl.ds(off, CF)], gidx)
        pltpu.async_copy(s_hbm.at[gidx], rsv, sem).wait()
        pltpu.async_copy(p0_hbm.at[gidx], r0v, sem).wait()
        pltpu.async_copy(p1_hbm.at[gidx], r1v, sem).wait()

        def row(i, carry):
            for j in range(DSEG):
                sl = pl.ds(j * 16, 16)
                rsv[i, sl] = (rsv[i, sl] + r0v[i, sl] + r1v[i, sl]) * 0.25
            return carry

        lax.fori_loop(0, CF, row, 0)
        pltpu.sync_copy(rsv, out_hbm.at[pl.ds(off, CF)])


def kernel(node_emb, edge_index, edge_weight, cas_idx, user_idx):
    src = edge_index[0]
    dst = edge_index[1]
    wbits = lax.bitcast_convert_type(edge_weight, jnp.int32)
    cap0 = NS * K0 * C
    cap1 = NS * K1 * C

    def part(x):
        xp = jnp.concatenate(
            [x, jnp.zeros((cap0 + cap1 - E,), jnp.int32)])
        a0 = xp[:cap0].reshape(NS, K0, C)
        a1 = xp[cap0:].reshape(NS, K1, C)
        a1 = jnp.pad(a1, ((0, 0), (0, K0 - K1), (0, 0)))
        return jnp.concatenate([a0, a1], axis=0)

    ed = jnp.stack([part(src), part(dst), part(wbits)], axis=2)
    e0 = jnp.zeros((N_PAD, D), jnp.float32).at[:N].set(node_emb)
    g = jnp.concatenate([cas_idx, user_idx + N_CAS]).astype(jnp.int32)

    p0, p1 = _edge_layer(e0, ed)
    e1, s1 = _combine(p0, p1, e0)
    p0, p1 = _edge_layer(e1, ed)
    e2, s2 = _combine(p0, p1, s1)
    p0, p1 = _edge_layer(e2, ed)
    return _final_gather(p0, p1, s2, g)


# speedup vs baseline: 1.8874x; 1.0195x over previous
import functools

import jax
import jax.numpy as jnp
from jax import lax
from jax.experimental import pallas as pl
from jax.experimental.pallas import tpu as pltpu
from jax.experimental.pallas import tpu_sc as plsc

N_CAS = 2000
N_USER = 8000
N = N_CAS + N_USER
E = 320000
D = 128
B = 4096
LAYERS = 3

NC = 2
NS = 16
NW = NC * NS
C = 96
N_PAD = 10240
ROWS_PER_SUB = N_PAD // NS
ROWS_PER_W = N_PAD // NW
K0 = 168
K1 = 42
KMAX = K0
OUT_ROWS_PER_W = 2 * B // NW
CF = 128
DSEG = D // 16

_mesh = plsc.VectorSubcoreMesh(core_axis_name="c", subcore_axis_name="s")


def _wid():
    return lax.axis_index("c") * NS + lax.axis_index("s")


@functools.partial(
    pl.kernel,
    out_type=(
        jax.ShapeDtypeStruct((N_PAD, D), jnp.float32),
        jax.ShapeDtypeStruct((N_PAD, D), jnp.float32),
    ),
    mesh=_mesh,
    scratch_types=(
        [pltpu.VMEM_SHARED((N_PAD, D), jnp.float32)]
        + [pltpu.VMEM((3, C), jnp.int32) for _ in range(6)]
        + [pltpu.VMEM((C, D), jnp.float32) for _ in range(3)]
        + [pltpu.SemaphoreType.DMA] * 12
    ),
    compiler_params=pltpu.CompilerParams(use_tc_tiling_on_sc=False,
                                        needs_layout_passes=False),
)
def _edge_layer(e_hbm, ed_hbm, p0_out, p1_out, acc,
                x0, x1, x2, x3, x4, x5,
                rb0, rb1, rb2,
                sg0, sg1, sg2, ss0, ss1, ss2,
                si0, si1, si2, si3, si4, si5):
    c = lax.axis_index("c")
    s = lax.axis_index("s")
    wid = c * NS + s
    r0 = s * ROWS_PER_SUB
    idxs = (x0, x1, x2, x3, x4, x5)
    rows = (rb0, rb1, rb2)
    gsem = (sg0, sg1, sg2)
    ssem = (ss0, ss1, ss2)
    isem = (si0, si1, si2, si3, si4, si5)
    nch = jnp.where(c == 0, K0, K1)
    ngrp = jnp.where(c == 0, K0 // 6, K1 // 6)

    def zfill(i, carry):
        for j in range(DSEG):
            rb0[i, pl.ds(j * 16, 16)] = jnp.zeros((16,), jnp.float32)
        return carry

    lax.fori_loop(0, 64, zfill, 0)

    def zcp(t, carry):
        pltpu.sync_copy(rb0.at[pl.ds(0, 64)], acc.at[pl.ds(r0 + t * 64, 64)])
        return carry

    lax.fori_loop(0, ROWS_PER_SUB // 64, zcp, 0)

    for j in range(4):
        pltpu.async_copy(ed_hbm.at[wid, j], idxs[j], isem[j])
    pltpu.make_async_copy(ed_hbm.at[wid, 0], idxs[0], isem[0]).wait()
    pltpu.async_copy(e_hbm.at[idxs[0].at[0]], rb0, sg0)
    pltpu.make_async_copy(ed_hbm.at[wid, 0], idxs[1], isem[1]).wait()
    pltpu.async_copy(e_hbm.at[idxs[1].at[0]], rb1, sg1)
    plsc.subcore_barrier()

    def do_chunk(k, i):
        b = i % 3
        b2 = (i + 2) % 3
        q = i % 6
        q2 = (i + 2) % 6
        q4 = (i + 4) % 6
        rcur = rows[b]
        xq = idxs[q]
        pltpu.make_async_copy(e_hbm.at[pl.ds(0, C)], rcur, gsem[b]).wait()

        def grp(gi, carry):
            w16 = plsc.bitcast(xq[2, pl.ds(gi * 16, 16)], jnp.float32)
            base_row = gi * 16
            for l in range(16):
                w = w16[l]
                for j in range(DSEG):
                    sl = pl.ds(j * 16, 16)
                    rcur[base_row + l, sl] = rcur[base_row + l, sl] * w
            return carry

        lax.fori_loop(0, 3, grp, 0)

        @pl.when(k >= 1)
        def _():
            pltpu.make_async_copy(e_hbm.at[pl.ds(0, C)], rows[b2],
                                  ssem[b2]).wait()

        @pl.when(k + 2 < nch)
        def _():
            pltpu.make_async_copy(ed_hbm.at[wid, 0], idxs[q2],
                                  isem[q2]).wait()
            pltpu.async_copy(e_hbm.at[idxs[q2].at[0]], rows[b2], gsem[b2])

        @pl.when(k + 4 < nch)
        def _():
            pltpu.async_copy(ed_hbm.at[wid, k + 4], idxs[q4], isem[q4])

        lax.fori_loop(3, 6, grp, 0)
        pltpu.async_copy(rcur, acc.at[xq.at[1]], ssem[b], add=True)

    def grp6(g, carry):
        k = g * 6
        for i in range(6):
            do_chunk(k + i, i)
        return carry

    lax.fori_loop(0, ngrp, grp6, 0)
    pltpu.make_async_copy(e_hbm.at[pl.ds(0, C)], rb2, ss2).wait()
    plsc.subcore_barrier()

    def wb_core(p_out):
        npc = ROWS_PER_SUB // 64
        pltpu.async_copy(acc.at[pl.ds(r0, 64)], rb0.at[pl.ds(0, 64)], sg0)

        def wb(t, carry):
            even = t % 2 == 0
            @pl.when(even)
            def _():
                pltpu.make_async_copy(acc.at[pl.ds(r0, 64)],
                                      rb0.at[pl.ds(0, 64)], sg0).wait()
                @pl.when(t + 1 < npc)
                def _():
                    pltpu.async_copy(acc.at[pl.ds(r0 + (t + 1) * 64, 64)],
                                     rb1.at[pl.ds(0, 64)], sg1)
                pltpu.sync_copy(rb0.at[pl.ds(0, 64)],
                                p_out.at[pl.ds(r0 + t * 64, 64)])
            @pl.when(jnp.logical_not(even))
            def _():
                pltpu.make_async_copy(acc.at[pl.ds(r0, 64)],
                                      rb1.at[pl.ds(0, 64)], sg1).wait()
                @pl.when(t + 1 < npc)
                def _():
                    pltpu.async_copy(acc.at[pl.ds(r0 + (t + 1) * 64, 64)],
                                     rb0.at[pl.ds(0, 64)], sg0)
                pltpu.sync_copy(rb1.at[pl.ds(0, 64)],
                                p_out.at[pl.ds(r0 + t * 64, 64)])
            return carry

        lax.fori_loop(0, npc, wb, 0)

    @pl.when(c == 0)
    def _():
        wb_core(p0_out)

    @pl.when(c == 1)
    def _():
        wb_core(p1_out)


@functools.partial(
    pl.kernel,
    out_type=(
        jax.ShapeDtypeStruct((N_PAD, D), jnp.float32),
        jax.ShapeDtypeStruct((N_PAD, D), jnp.float32),
    ),
    mesh=_mesh,
    scratch_types=[
        pltpu.VMEM((ROWS_PER_W, D), jnp.float32),
        pltpu.VMEM((ROWS_PER_W, D), jnp.float32),
        pltpu.VMEM((ROWS_PER_W, D), jnp.float32),
    ],
)
def _combine(p0_hbm, p1_hbm, s_hbm, e_out, s_out, p0v, p1v, sv):
    wid = _wid()
    r0 = wid * ROWS_PER_W
    pltpu.sync_copy(p0_hbm.at[pl.ds(r0, ROWS_PER_W)], p0v)
    pltpu.sync_copy(p1_hbm.at[pl.ds(r0, ROWS_PER_W)], p1v)
    pltpu.sync_copy(s_hbm.at[pl.ds(r0, ROWS_PER_W)], sv)

    def row(i, carry):
        for j in range(DSEG):
            sl = pl.ds(j * 16, 16)
            e = p0v[i, sl] + p1v[i, sl]
            p0v[i, sl] = e
            sv[i, sl] = sv[i, sl] + e
        return carry

    lax.fori_loop(0, ROWS_PER_W, row, 0)
    pltpu.sync_copy(p0v, e_out.at[pl.ds(r0, ROWS_PER_W)])
    pltpu.sync_copy(sv, s_out.at[pl.ds(r0, ROWS_PER_W)])


@functools.partial(
    pl.kernel,
    out_type=jax.ShapeDtypeStruct((2 * B, D), jnp.float32),
    mesh=_mesh,
    scratch_types=[
        pltpu.VMEM((CF,), jnp.int32),
        pltpu.VMEM((CF, D), jnp.float32),
        pltpu.VMEM((CF, D), jnp.float32),
        pltpu.VMEM((CF, D), jnp.float32),
        pltpu.SemaphoreType.DMA,
    ],
)
def _final_gather(p0_hbm, p1_hbm, s_hbm, g_hbm, out_hbm,
                  gidx, rsv, r0v, r1v, sem):
    wid = _wid()
    for t in range(OUT_ROWS_PER_W // CF):
        off = wid * OUT_ROWS_PER_W + t * CF
        pltpu.sync_copy(g_hbm.at[pl.ds(off, CF)], gidx)
        pltpu.async_copy(s_hbm.at[gidx], rsv, sem).wait()
        pltpu.async_copy(p0_hbm.at[gidx], r0v, sem).wait()
        pltpu.async_copy(p1_hbm.at[gidx], r1v, sem).wait()

        def row(i, carry):
            for j in range(DSEG):
                sl = pl.ds(j * 16, 16)
                rsv[i, sl] = (rsv[i, sl] + r0v[i, sl] + r1v[i, sl]) * 0.25
            return carry

        lax.fori_loop(0, CF, row, 0)
        pltpu.sync_copy(rsv, out_hbm.at[pl.ds(off, CF)])


def kernel(node_emb, edge_index, edge_weight, cas_idx, user_idx):
    src = edge_index[0]
    dst = edge_index[1]
    wbits = lax.bitcast_convert_type(edge_weight, jnp.int32)
    cap0 = NS * K0 * C
    cap1 = NS * K1 * C

    def part(x):
        xp = jnp.concatenate(
            [x, jnp.zeros((cap0 + cap1 - E,), jnp.int32)])
        a0 = xp[:cap0].reshape(NS, K0, C)
        a1 = xp[cap0:].reshape(NS, K1, C)
        a1 = jnp.pad(a1, ((0, 0), (0, K0 - K1), (0, 0)))
        return jnp.concatenate([a0, a1], axis=0)

    ed = jnp.stack([part(src), part(dst), part(wbits)], axis=2)
    e0 = jnp.zeros((N_PAD, D), jnp.float32).at[:N].set(node_emb)
    g = jnp.concatenate([cas_idx, user_idx + N_CAS]).astype(jnp.int32)

    p0, p1 = _edge_layer(e0, ed)
    e1, s1 = _combine(p0, p1, e0)
    p0, p1 = _edge_layer(e1, ed)
    e2, s2 = _combine(p0, p1, s1)
    p0, p1 = _edge_layer(e2, ed)
    return _final_gather(p0, p1, s2, g)


# speedup vs baseline: 1.9000x; 1.0067x over previous
import functools

import jax
import jax.numpy as jnp
from jax import lax
from jax.experimental import pallas as pl
from jax.experimental.pallas import tpu as pltpu
from jax.experimental.pallas import tpu_sc as plsc

N_CAS = 2000
N_USER = 8000
N = N_CAS + N_USER
E = 320000
D = 128
B = 4096
LAYERS = 3

NC = 2
NS = 16
NW = NC * NS
C = 96
N_PAD = 10240
ROWS_PER_SUB = N_PAD // NS
ROWS_PER_W = N_PAD // NW
K0 = 168
K1 = 42
KMAX = K0
OUT_ROWS_PER_W = 2 * B // NW
CF = 128
DSEG = D // 16

_mesh = plsc.VectorSubcoreMesh(core_axis_name="c", subcore_axis_name="s")


def _wid():
    return lax.axis_index("c") * NS + lax.axis_index("s")


@functools.partial(
    pl.kernel,
    out_type=(
        jax.ShapeDtypeStruct((N_PAD, D), jnp.float32),
        jax.ShapeDtypeStruct((N_PAD, D), jnp.float32),
    ),
    mesh=_mesh,
    scratch_types=(
        [pltpu.VMEM_SHARED((N_PAD, D), jnp.float32)]
        + [pltpu.VMEM((3, C), jnp.int32) for _ in range(6)]
        + [pltpu.VMEM((C, D), jnp.float32) for _ in range(3)]
        + [pltpu.SemaphoreType.DMA] * 12
    ),
    compiler_params=pltpu.CompilerParams(use_tc_tiling_on_sc=False,
                                        needs_layout_passes=False),
)
def _edge_layer(e_hbm, ed_hbm, p0_out, p1_out, acc,
                x0, x1, x2, x3, x4, x5,
                rb0, rb1, rb2,
                sg0, sg1, sg2, ss0, ss1, ss2,
                si0, si1, si2, si3, si4, si5):
    c = lax.axis_index("c")
    s = lax.axis_index("s")
    wid = c * NS + s
    r0 = s * ROWS_PER_SUB
    idxs = (x0, x1, x2, x3, x4, x5)
    rows = (rb0, rb1, rb2)
    gsem = (sg0, sg1, sg2)
    ssem = (ss0, ss1, ss2)
    isem = (si0, si1, si2, si3, si4, si5)
    nch = jnp.where(c == 0, K0, K1)
    ngrp = jnp.where(c == 0, K0 // 6, K1 // 6)

    def zfill(i, carry):
        for j in range(DSEG):
            rb0[i, pl.ds(j * 16, 16)] = jnp.zeros((16,), jnp.float32)
        return carry

    lax.fori_loop(0, 64, zfill, 0)

    def zcp(t, carry):
        pltpu.sync_copy(rb0.at[pl.ds(0, 64)], acc.at[pl.ds(r0 + t * 64, 64)])
        return carry

    lax.fori_loop(0, ROWS_PER_SUB // 64, zcp, 0)

    for j in range(4):
        pltpu.async_copy(ed_hbm.at[wid, j], idxs[j], isem[j])
    pltpu.make_async_copy(ed_hbm.at[wid, 0], idxs[0], isem[0]).wait()
    pltpu.async_copy(e_hbm.at[idxs[0].at[0]], rb0, sg0)
    pltpu.make_async_copy(ed_hbm.at[wid, 0], idxs[1], isem[1]).wait()
    pltpu.async_copy(e_hbm.at[idxs[1].at[0]], rb1, sg1)
    plsc.subcore_barrier()

    def do_chunk(k, i):
        b = i % 3
        b2 = (i + 2) % 3
        q = i % 6
        q2 = (i + 2) % 6
        q4 = (i + 4) % 6
        rcur = rows[b]
        xq = idxs[q]
        pltpu.make_async_copy(e_hbm.at[pl.ds(0, C)], rcur, gsem[b]).wait()

        def grp(gi, carry):
            w16 = plsc.bitcast(xq[2, pl.ds(gi * 16, 16)], jnp.float32)
            base_row = gi * 16
            for l in range(16):
                w = w16[l]
                for j in range(DSEG):
                    sl = pl.ds(j * 16, 16)
                    rcur[base_row + l, sl] = rcur[base_row + l, sl] * w
            return carry

        lax.fori_loop(0, 3, grp, 0)

        @pl.when(k >= 1)
        def _():
            pltpu.make_async_copy(e_hbm.at[pl.ds(0, C)], rows[b2],
                                  ssem[b2]).wait()

        @pl.when(k + 2 < nch)
        def _():
            pltpu.make_async_copy(ed_hbm.at[wid, 0], idxs[q2],
                                  isem[q2]).wait()
            pltpu.async_copy(e_hbm.at[idxs[q2].at[0]], rows[b2], gsem[b2])

        @pl.when(k + 4 < nch)
        def _():
            pltpu.async_copy(ed_hbm.at[wid, k + 4], idxs[q4], isem[q4])

        lax.fori_loop(3, 6, grp, 0)
        pltpu.async_copy(rcur, acc.at[xq.at[1]], ssem[b], add=True)

    def grp6(g, carry):
        k = g * 6
        for i in range(6):
            do_chunk(k + i, i)
        return carry

    lax.fori_loop(0, ngrp, grp6, 0)
    pltpu.make_async_copy(e_hbm.at[pl.ds(0, C)], rb2, ss2).wait()
    plsc.subcore_barrier()

    def wb_core(p_out):
        npc = ROWS_PER_SUB // 64
        pltpu.async_copy(acc.at[pl.ds(r0, 64)], rb0.at[pl.ds(0, 64)], sg0)

        def wb(t, carry):
            even = t % 2 == 0
            @pl.when(even)
            def _():
                pltpu.make_async_copy(acc.at[pl.ds(r0, 64)],
                                      rb0.at[pl.ds(0, 64)], sg0).wait()
                @pl.when(t + 1 < npc)
                def _():
                    pltpu.async_copy(acc.at[pl.ds(r0 + (t + 1) * 64, 64)],
                                     rb1.at[pl.ds(0, 64)], sg1)
                pltpu.sync_copy(rb0.at[pl.ds(0, 64)],
                                p_out.at[pl.ds(r0 + t * 64, 64)])
            @pl.when(jnp.logical_not(even))
            def _():
                pltpu.make_async_copy(acc.at[pl.ds(r0, 64)],
                                      rb1.at[pl.ds(0, 64)], sg1).wait()
                @pl.when(t + 1 < npc)
                def _():
                    pltpu.async_copy(acc.at[pl.ds(r0 + (t + 1) * 64, 64)],
                                     rb0.at[pl.ds(0, 64)], sg0)
                pltpu.sync_copy(rb1.at[pl.ds(0, 64)],
                                p_out.at[pl.ds(r0 + t * 64, 64)])
            return carry

        lax.fori_loop(0, npc, wb, 0)

    @pl.when(c == 0)
    def _():
        wb_core(p0_out)

    @pl.when(c == 1)
    def _():
        wb_core(p1_out)


@functools.partial(
    pl.kernel,
    out_type=(
        jax.ShapeDtypeStruct((N_PAD, D), jnp.float32),
        jax.ShapeDtypeStruct((N_PAD, D), jnp.float32),
    ),
    mesh=_mesh,
    scratch_types=[
        pltpu.VMEM((ROWS_PER_W, D), jnp.float32),
        pltpu.VMEM((ROWS_PER_W, D), jnp.float32),
        pltpu.VMEM((ROWS_PER_W, D), jnp.float32),
    ],
)
def _combine(p0_hbm, p1_hbm, s_hbm, e_out, s_out, p0v, p1v, sv):
    wid = _wid()
    r0 = wid * ROWS_PER_W
    pltpu.sync_copy(p0_hbm.at[pl.ds(r0, ROWS_PER_W)], p0v)
    pltpu.sync_copy(p1_hbm.at[pl.ds(r0, ROWS_PER_W)], p1v)
    pltpu.sync_copy(s_hbm.at[pl.ds(r0, ROWS_PER_W)], sv)

    def row(i, carry):
        for j in range(DSEG):
            sl = pl.ds(j * 16, 16)
            e = p0v[i, sl] + p1v[i, sl]
            p0v[i, sl] = e
            sv[i, sl] = sv[i, sl] + e
        return carry

    lax.fori_loop(0, ROWS_PER_W, row, 0)
    pltpu.sync_copy(p0v, e_out.at[pl.ds(r0, ROWS_PER_W)])
    pltpu.sync_copy(sv, s_out.at[pl.ds(r0, ROWS_PER_W)])


@functools.partial(
    pl.kernel,
    out_type=jax.ShapeDtypeStruct((2 * B, D), jnp.float32),
    mesh=_mesh,
    scratch_types=[
        pltpu.VMEM((CF,), jnp.int32),
        pltpu.VMEM((CF, D), jnp.float32),
        pltpu.VMEM((CF, D), jnp.float32),
        pltpu.VMEM((CF, D), jnp.float32),
        pltpu.SemaphoreType.DMA,
    ],
)
def _final_gather(p0_hbm, p1_hbm, s_hbm, g_hbm, out_hbm,
                  gidx, rsv, r0v, r1v, sem):
    wid = _wid()
    for t in range(OUT_ROWS_PER_W // CF):
        off = wid * OUT_ROWS_PER_W + t * CF
        pltpu.sync_copy(g_hbm.at[pl.ds(off, CF)], gidx)
        pltpu.async_copy(s_hbm.at[gidx], rsv, sem)
        pltpu.async_copy(p0_hbm.at[gidx], r0v, sem)
        pltpu.async_copy(p1_hbm.at[gidx], r1v, sem)
        pltpu.make_async_copy(s_hbm.at[pl.ds(0, CF)], rsv, sem).wait()
        pltpu.make_async_copy(s_hbm.at[pl.ds(0, CF)], r0v, sem).wait()
        pltpu.make_async_copy(s_hbm.at[pl.ds(0, CF)], r1v, sem).wait()

        def row(i, carry):
            for j in range(DSEG):
                sl = pl.ds(j * 16, 16)
                rsv[i, sl] = (rsv[i, sl] + r0v[i, sl] + r1v[i, sl]) * 0.25
            return carry

        lax.fori_loop(0, CF, row, 0)
        pltpu.sync_copy(rsv, out_hbm.at[pl.ds(off, CF)])


def kernel(node_emb, edge_index, edge_weight, cas_idx, user_idx):
    src = edge_index[0]
    dst = edge_index[1]
    wbits = lax.bitcast_convert_type(edge_weight, jnp.int32)
    cap0 = NS * K0 * C
    cap1 = NS * K1 * C

    def part(x):
        xp = jnp.concatenate(
            [x, jnp.zeros((cap0 + cap1 - E,), jnp.int32)])
        a0 = xp[:cap0].reshape(NS, K0, C)
        a1 = xp[cap0:].reshape(NS, K1, C)
        a1 = jnp.pad(a1, ((0, 0), (0, K0 - K1), (0, 0)))
        return jnp.concatenate([a0, a1], axis=0)

    ed = jnp.stack([part(src), part(dst), part(wbits)], axis=2)
    e0 = jnp.zeros((N_PAD, D), jnp.float32).at[:N].set(node_emb)
    g = jnp.concatenate([cas_idx, user_idx + N_CAS]).astype(jnp.int32)

    p0, p1 = _edge_layer(e0, ed)
    e1, s1 = _combine(p0, p1, e0)
    p0, p1 = _edge_layer(e1, ed)
    e2, s2 = _combine(p0, p1, s1)
    p0, p1 = _edge_layer(e2, ed)
    return _final_gather(p0, p1, s2, g)
